# Initial kernel scaffold; baseline (speedup 1.0000x reference)
import jax
import jax.numpy as jnp
from jax.experimental import pallas as pl


def kernel(feat_edit, feat_ori, feat_2d):
    raise NotImplementedError("write your pallas kernel here")


# speedup vs baseline: 1.6383x; 1.6383x over previous
import jax
import jax.numpy as jnp
from jax.experimental import pallas as pl

_K, _PAD, _STRIDE = 5, 1, 2
_H = 128
_OH = (_H + 2 * _PAD - _K) // _STRIDE + 1
_L = _OH * _OH
_C25 = 64 * _K * _K
_LP = 4096
_CP = 1664
_TI = 1024
_TJ = 2048


def _patch_matrix(x):
    xc = x[0]
    xp = jnp.pad(xc, ((0, 0), (_PAD, _PAD), (_PAD, _PAD)))
    span = _STRIDE * (_OH - 1) + 1
    cols = []
    for i in range(_K):
        for j in range(_K):
            cols.append(xp[:, i:i + span:_STRIDE, j:j + span:_STRIDE])
    p = jnp.stack(cols, axis=1)
    return p.reshape(_C25, _L)


def _body(k_ref, q_ref, o_ref):
    i = pl.program_id(1)
    kb = k_ref[...].astype(jnp.float32)
    inv = jax.lax.rsqrt(jnp.maximum(jnp.sum(kb * kb, axis=1, keepdims=True), 1e-24))
    kn = (kb * inv).astype(jnp.bfloat16)
    r = jnp.dot(kn, q_ref[...], preferred_element_type=jnp.float32)
    m = jnp.max(r, axis=0, keepdims=True)
    acc = jnp.where(i == 0, jnp.full_like(m, -jnp.inf), o_ref[...])
    o_ref[...] = jnp.maximum(acc, m)

    @pl.when(i == pl.num_programs(1) - 1)
    def _():
        qf = q_ref[...].astype(jnp.float32)
        qn = jnp.sqrt(jnp.sum(qf * qf, axis=0, keepdims=True))
        o_ref[...] = o_ref[...] / jnp.maximum(qn, 1e-12)


def kernel(feat_edit, feat_ori, feat_2d):
    del feat_2d
    q_mat = _patch_matrix(feat_ori)
    k_mat = _patch_matrix(feat_edit).T

    k_mat = jnp.pad(k_mat, ((0, 0), (0, _CP - _C25)))
    k_mat = jnp.pad(k_mat, ((0, _LP - _L), (0, 0)), mode="edge")
    q_mat = jnp.pad(q_mat, ((0, _CP - _C25), (0, _LP - _L)))
    k_bf = k_mat.astype(jnp.bfloat16)
    q_bf = q_mat.astype(jnp.bfloat16)

    out = pl.pallas_call(
        _body,
        grid=(_LP // _TJ, _LP // _TI),
        in_specs=[
            pl.BlockSpec((_TI, _CP), lambda j, i: (i, 0)),
            pl.BlockSpec((_CP, _TJ), lambda j, i: (0, j)),
        ],
        out_specs=pl.BlockSpec((1, _TJ), lambda j, i: (0, j)),
        out_shape=jax.ShapeDtypeStruct((1, _LP), jnp.float32),
    )(k_bf, q_bf)

    return out[0, :_L].reshape(1, 1, _OH, _OH)


# speedup vs baseline: 4.0503x; 2.4723x over previous
import jax
import jax.numpy as jnp
from jax.experimental import pallas as pl

_K, _PAD, _STRIDE = 5, 1, 2
_H = 128
_OH = (_H + 2 * _PAD - _K) // _STRIDE + 1
_L = _OH * _OH
_C25 = 64 * _K * _K
_LP = 4096
_CP = 1664
_TI = 1024
_TJ = 2048


def _patch_matrix(x):
    xc = x[0]
    xp = jnp.pad(xc, ((0, 0), (_PAD, _PAD), (_PAD, _PAD)))
    ph = [[xp[:, a::2, b::2] for b in range(2)] for a in range(2)]
    cols = []
    for i in range(_K):
        for j in range(_K):
            p = ph[i % 2][j % 2]
            cols.append(p[:, i // 2:i // 2 + _OH, j // 2:j // 2 + _OH])
    p = jnp.stack(cols, axis=0)
    return p.reshape(_C25, _L)


def _body(k_ref, q_ref, o_ref):
    i = pl.program_id(1)
    kb = k_ref[...].astype(jnp.float32)
    inv = jax.lax.rsqrt(jnp.maximum(jnp.sum(kb * kb, axis=0, keepdims=True), 1e-24))
    kn = (kb * inv).astype(jnp.bfloat16)
    r = jax.lax.dot_general(
        kn, q_ref[...],
        dimension_numbers=(((0,), (0,)), ((), ())),
        preferred_element_type=jnp.float32,
    )
    m = jnp.max(r, axis=0, keepdims=True)
    acc = jnp.where(i == 0, jnp.full_like(m, -jnp.inf), o_ref[...])
    o_ref[...] = jnp.maximum(acc, m)

    @pl.when(i == pl.num_programs(1) - 1)
    def _():
        qf = q_ref[...].astype(jnp.float32)
        qn = jnp.sqrt(jnp.sum(qf * qf, axis=0, keepdims=True))
        o_ref[...] = o_ref[...] / jnp.maximum(qn, 1e-12)


def kernel(feat_edit, feat_ori, feat_2d):
    del feat_2d
    q_mat = _patch_matrix(feat_ori)
    k_mat = _patch_matrix(feat_edit)

    k_mat = jnp.pad(k_mat, ((0, _CP - _C25), (0, 0)))
    k_mat = jnp.pad(k_mat, ((0, 0), (0, _LP - _L)), mode="edge")
    q_mat = jnp.pad(q_mat, ((0, _CP - _C25), (0, _LP - _L)))
    k_bf = k_mat.astype(jnp.bfloat16)
    q_bf = q_mat.astype(jnp.bfloat16)

    out = pl.pallas_call(
        _body,
        grid=(_LP // _TJ, _LP // _TI),
        in_specs=[
            pl.BlockSpec((_CP, _TI), lambda j, i: (0, i)),
            pl.BlockSpec((_CP, _TJ), lambda j, i: (0, j)),
        ],
        out_specs=pl.BlockSpec((1, _TJ), lambda j, i: (0, j)),
        out_shape=jax.ShapeDtypeStruct((1, _LP), jnp.float32),
    )(k_bf, q_bf)

    return out[0, :_L].reshape(1, 1, _OH, _OH)
